# Initial kernel scaffold; baseline (speedup 1.0000x reference)
import jax
import jax.numpy as jnp
from jax.experimental import pallas as pl


def kernel(probs, k):
    raise NotImplementedError("write your pallas kernel here")


# speedup vs baseline: 25.9755x; 25.9755x over previous
import jax
import jax.numpy as jnp
from jax import lax
from jax.experimental import pallas as pl
from jax.experimental.pallas import tpu as pltpu

_TEMP = 0.7
_TOPP = 0.9
_EPS = 1e-20
_K = 50
_INTMAX = 2147483647


def _topk_body(p_ref, kth_ref):
    p = p_ref[...]
    xi = lax.bitcast_convert_type(p, jnp.int32)
    ximax = jnp.max(xi, axis=-1, keepdims=True)

    def cond(c):
        lo, hi = c
        return jnp.any(hi - lo > 1)

    def body(c):
        lo, hi = c
        mid = lo + lax.div(hi - lo + 1, 2)
        cnt = jnp.sum((xi >= mid).astype(jnp.int32), axis=-1, keepdims=True)
        ok = cnt >= _K
        return jnp.where(ok, mid, lo), jnp.where(ok, hi, mid)

    lo, _ = lax.while_loop(cond, body, (jnp.zeros_like(ximax), ximax + 1))
    kth_ref[...] = lax.bitcast_convert_type(lo, jnp.float32)


def _nucleus_body(p1_ref, out_ref):
    p1 = p1_ref[...]
    xi = lax.bitcast_convert_type(p1, jnp.int32)
    ximax = jnp.max(xi, axis=-1, keepdims=True)
    minpos = jnp.min(jnp.where(xi > 0, xi, _INTMAX), axis=-1, keepdims=True)

    def cond(c):
        lo, hi, th = c
        return jnp.any(hi - lo > 1)

    def body(c):
        lo, hi, th = c
        mid = lo + lax.div(hi - lo + 1, 2)
        mv = jnp.min(jnp.where(xi >= mid, xi, _INTMAX), axis=-1, keepdims=True)
        sgt = jnp.sum(jnp.where(xi > mv, p1, 0.0), axis=-1, keepdims=True)
        qv = jnp.max(jnp.where(xi == mv, p1, 0.0), axis=-1, keepdims=True)
        ok = (sgt + qv <= _TOPP) | (mv >= ximax)
        return (jnp.where(ok, lo, mid), jnp.where(ok, mid, hi),
                jnp.where(ok, mv, th))

    _, _, thb = lax.while_loop(cond, body, (minpos - 1, ximax, ximax))

    keep = xi >= thb
    s2 = jnp.sum(jnp.where(keep, p1, 0.0), axis=-1, keepdims=True)
    out_ref[...] = jnp.where(keep, p1 / s2, 0.0)


def kernel(probs, k):
    del k
    B, V = probs.shape
    BR = 8
    row_spec = pl.BlockSpec((BR, V), lambda i: (i, 0))

    logits = jnp.log(probs + _EPS)
    logits = logits / _TEMP
    p = jax.nn.softmax(logits, axis=-1)

    kth = pl.pallas_call(
        _topk_body,
        grid=(B // BR,),
        in_specs=[row_spec],
        out_specs=pl.BlockSpec((BR, 1), lambda i: (i, 0)),
        out_shape=jax.ShapeDtypeStruct((B, 1), jnp.float32),
    )(p)

    pm = jnp.where(p >= kth, p, jnp.zeros_like(p))
    p1 = pm / jnp.sum(pm, axis=-1, keepdims=True)

    return pl.pallas_call(
        _nucleus_body,
        grid=(B // BR,),
        in_specs=[row_spec],
        out_specs=row_spec,
        out_shape=jax.ShapeDtypeStruct((B, V), jnp.float32),
    )(p1)


# speedup vs baseline: 48.3694x; 1.8621x over previous
import jax
import jax.numpy as jnp
from jax import lax
from jax.experimental import pallas as pl
from jax.experimental.pallas import tpu as pltpu

_TEMP = 0.7
_TOPP = 0.9
_EPS = 1e-20
_K = 50
_INTMAX = 2147483647
_CH = 32
_NCH = 3125


def _topk_body(p3_ref, kth_ref):
    x3 = p3_ref[...]
    xi3 = lax.bitcast_convert_type(x3, jnp.int32)
    cm = jnp.max(xi3, axis=1)
    ximax = jnp.max(cm, axis=-1, keepdims=True)

    def c_cond(c):
        lo, hi = c
        return jnp.any(hi - lo > 1)

    def c_body(c):
        lo, hi = c
        mid = lo + lax.div(hi - lo + 1, 2)
        cnt = jnp.sum((cm >= mid).astype(jnp.int32), axis=-1, keepdims=True)
        ok = cnt >= _K
        return jnp.where(ok, mid, lo), jnp.where(ok, hi, mid)

    c50, _ = lax.while_loop(c_cond, c_body, (jnp.zeros_like(ximax), ximax + 1))

    def cond(c):
        lo, hi = c
        return jnp.any(hi - lo > 1)

    def body(c):
        lo, hi = c
        mid = lo + lax.div(hi - lo + 1, 2)
        m = (xi3 >= mid[:, :, None]).astype(jnp.int32)
        cnt = jnp.sum(jnp.sum(m, axis=2), axis=-1, keepdims=True)
        ok = cnt >= _K
        return jnp.where(ok, mid, lo), jnp.where(ok, hi, mid)

    lo, _ = lax.while_loop(cond, body, (c50, ximax + 1))
    kth_ref[...] = lax.bitcast_convert_type(lo, jnp.float32)


def _nucleus_body(p_ref, kth_ref, s1_ref, out_ref, p1_ref):
    p = p_ref[...]
    kth = kth_ref[...]
    s1 = s1_ref[...]
    p1 = jnp.where(p >= kth, p, 0.0) / s1
    p1_ref[...] = p1
    xi = lax.bitcast_convert_type(p1, jnp.int32)
    ximax = jnp.max(xi, axis=-1, keepdims=True)
    minpos = jnp.min(jnp.where(xi > 0, xi, _INTMAX), axis=-1, keepdims=True)

    def cond(c):
        lo, hi = c
        return jnp.any(hi - lo > 1)

    def body(c):
        lo, hi = c
        mid = lo + lax.div(hi - lo + 1, 2)
        sge = jnp.sum(jnp.where(xi >= mid, p1, 0.0), axis=-1, keepdims=True)
        ok = sge <= _TOPP
        return jnp.where(ok, lo, mid), jnp.where(ok, mid, hi)

    _, tstar = lax.while_loop(cond, body, (minpos - 1, ximax + 1))

    vc0 = jnp.min(jnp.where(xi >= tstar, xi, _INTMAX), axis=-1, keepdims=True)
    vc = jnp.where(vc0 == _INTMAX, ximax, vc0)
    vd = jnp.max(jnp.where((xi > 0) & (xi < vc), xi, 0), axis=-1, keepdims=True)
    sgt_d = jnp.sum(jnp.where(xi > vd, p1, 0.0), axis=-1, keepdims=True)
    qd = jnp.max(jnp.where(xi == vd, p1, 0.0), axis=-1, keepdims=True)
    thb = jnp.where(sgt_d + qd <= _TOPP, vd, vc)

    keep = xi >= thb
    s2 = jnp.sum(jnp.where(keep, p1, 0.0), axis=-1, keepdims=True)
    out_ref[...] = jnp.where(keep, p1 / s2, 0.0)


def kernel(probs, k):
    del k
    B, V = probs.shape

    logits = jnp.log(probs + _EPS)
    logits = logits / _TEMP
    p = jax.nn.softmax(logits, axis=-1)

    BRT = 32
    kth = pl.pallas_call(
        _topk_body,
        grid=(B // BRT,),
        in_specs=[pl.BlockSpec((BRT, _CH, _NCH), lambda i: (i, 0, 0))],
        out_specs=pl.BlockSpec((BRT, 1), lambda i: (i, 0)),
        out_shape=jax.ShapeDtypeStruct((B, 1), jnp.float32),
    )(p.reshape(B, _CH, _NCH))

    s1 = jnp.sum(jnp.where(p >= kth, p, jnp.zeros_like(p)),
                 axis=-1, keepdims=True)

    BRN = 16
    row_spec = pl.BlockSpec((BRN, V), lambda i: (i, 0))
    col_spec = pl.BlockSpec((BRN, 1), lambda i: (i, 0))
    return pl.pallas_call(
        _nucleus_body,
        grid=(B // BRN,),
        in_specs=[row_spec, col_spec, col_spec],
        out_specs=row_spec,
        out_shape=jax.ShapeDtypeStruct((B, V), jnp.float32),
        scratch_shapes=[pltpu.VMEM((BRN, V), jnp.float32)],
    )(p, kth, s1)


# speedup vs baseline: 50.4410x; 1.0428x over previous
import functools

import jax
import jax.numpy as jnp
from jax import lax
from jax.experimental import pallas as pl
from jax.experimental.pallas import tpu as pltpu
from jax.experimental.pallas import tpu_sc as plsc

_TEMP = 0.7
_TOPP = 0.9
_EPS = 1e-20
_K = 50
_INTMAX = 2147483647


_V = 100000
_NG = 250
_GV = 25


def _sc_topk_body(probs_hbm, out_hbm, rowbuf, gmxv, kthbuf, gmx_smem):
    c = lax.axis_index("c")
    s = lax.axis_index("s")
    wid = s * 2 + c
    lane = lax.iota(jnp.int32, 16)
    kth = jnp.zeros((16,), jnp.float32)

    for j in range(2):
        row = wid * 2 + j
        pltpu.sync_copy(probs_hbm.at[row], rowbuf)

        def ga(g, carry):
            maxacc, gvec = carry
            base = g * (_GV * 16)
            gm = plsc.bitcast(rowbuf[pl.ds(base, 16)], jnp.int32)
            for jj in range(1, _GV):
                v = plsc.bitcast(rowbuf[pl.ds(base + jj * 16, 16)], jnp.int32)
                gm = jnp.maximum(gm, v)
            gs = jnp.max(gm)
            gmx_smem[g] = gs
            gvec = jnp.where(lane == g % 16, gs, gvec)

            @pl.when(g % 16 == 15)
            def _():
                gmxv[pl.ds((g // 16) * 16, 16)] = gvec

            return jnp.maximum(maxacc, gm), gvec

        maxacc, gvec = lax.fori_loop(
            0, _NG, ga,
            (jnp.full((16,), -2**31, jnp.int32), jnp.zeros((16,), jnp.int32)))
        gmxv[pl.ds((_NG // 16) * 16, 16)] = jnp.where(
            lane < _NG % 16, gvec, 0)
        m = jnp.max(maxacc)

        def b_cond(cs):
            lo, hi = cs
            return hi - lo > 1

        def b_body(cs):
            lo, hi = cs
            mid = lo + (hi - lo + 1) // 2
            vmid = jnp.full((16,), mid, jnp.int32)

            def cb(i, acc):
                return acc + (gmxv[pl.ds(i * 16, 16)] >= vmid).astype(jnp.int32)

            acc = lax.fori_loop(0, 16, cb, jnp.zeros((16,), jnp.int32))
            ok = jnp.sum(acc) >= _K
            return jnp.where(ok, mid, lo), jnp.where(ok, hi, mid)

        c50, _ = lax.while_loop(b_cond, b_body, (jnp.int32(0), m + 1))

        def f_body(cs):
            lo, hi = cs
            mid = lo + (hi - lo + 1) // 2
            vmid = jnp.full((16,), mid, jnp.int32)

            def fb(g, acc):
                def count_group(a):
                    base = g * (_GV * 16)
                    for jj in range(_GV):
                        v = plsc.bitcast(
                            rowbuf[pl.ds(base + jj * 16, 16)], jnp.int32)
                        a = a + (v >= vmid).astype(jnp.int32)
                    return a

                return lax.cond(gmx_smem[g] >= mid, count_group,
                                lambda a: a, acc)

            acc = lax.fori_loop(0, _NG, fb, jnp.zeros((16,), jnp.int32))
            ok = jnp.sum(acc) >= _K
            return jnp.where(ok, mid, lo), jnp.where(ok, hi, mid)

        kb, _ = lax.while_loop(b_cond, f_body, (c50, m + 1))
        kfv = plsc.bitcast(jnp.full((16,), kb, jnp.int32), jnp.float32)
        kth = jnp.where(lane == j, kfv, kth)

    kthbuf[...] = kth
    pltpu.sync_copy(kthbuf, out_hbm.at[wid])


def _sc_topk(probs):
    mesh = plsc.VectorSubcoreMesh(core_axis_name="c", subcore_axis_name="s")
    fn = functools.partial(
        pl.kernel,
        mesh=mesh,
        compiler_params=pltpu.CompilerParams(needs_layout_passes=False),
        out_type=jax.ShapeDtypeStruct((32, 16), jnp.float32),
        scratch_types=[
            pltpu.VMEM((_V,), jnp.float32),
            pltpu.VMEM((256,), jnp.int32),
            pltpu.VMEM((16,), jnp.float32),
            pltpu.SMEM((256,), jnp.int32),
        ],
    )(_sc_topk_body)
    return fn(probs)


def _nucleus_body(p_ref, kth_ref, s1_ref, out_ref, p1_ref):
    p = p_ref[...]
    kth = kth_ref[...]
    s1 = s1_ref[...]
    p1 = jnp.where(p >= kth, p, 0.0) / s1
    p1_ref[...] = p1
    xi = lax.bitcast_convert_type(p1, jnp.int32)
    ximax = jnp.max(xi, axis=-1, keepdims=True)
    minpos = jnp.min(jnp.where(xi > 0, xi, _INTMAX), axis=-1, keepdims=True)

    def cond(c):
        lo, hi = c
        return jnp.any(hi - lo > 1)

    def body(c):
        lo, hi = c
        mid = lo + lax.div(hi - lo + 1, 2)
        sge = jnp.sum(jnp.where(xi >= mid, p1, 0.0), axis=-1, keepdims=True)
        ok = sge <= _TOPP
        return jnp.where(ok, lo, mid), jnp.where(ok, mid, hi)

    _, tstar = lax.while_loop(cond, body, (minpos - 1, ximax + 1))

    vc0 = jnp.min(jnp.where(xi >= tstar, xi, _INTMAX), axis=-1, keepdims=True)
    vc = jnp.where(vc0 == _INTMAX, ximax, vc0)
    vd = jnp.max(jnp.where((xi > 0) & (xi < vc), xi, 0), axis=-1, keepdims=True)
    sgt_d = jnp.sum(jnp.where(xi > vd, p1, 0.0), axis=-1, keepdims=True)
    qd = jnp.max(jnp.where(xi == vd, p1, 0.0), axis=-1, keepdims=True)
    thb = jnp.where(sgt_d + qd <= _TOPP, vd, vc)

    keep = xi >= thb
    s2 = jnp.sum(jnp.where(keep, p1, 0.0), axis=-1, keepdims=True)
    out_ref[...] = jnp.where(keep, p1 / s2, 0.0)


def kernel(probs, k):
    del k
    B, V = probs.shape

    kthw = _sc_topk(probs)
    kthp = kthw[:, :2].reshape(B, 1)

    logits = jnp.log(probs + _EPS)
    logits = logits / _TEMP
    p = jax.nn.softmax(logits, axis=-1)

    T = jnp.max(jnp.where(probs == kthp, p, jnp.zeros_like(p)),
                axis=-1, keepdims=True)
    s1 = jnp.sum(jnp.where(p >= T, p, jnp.zeros_like(p)),
                 axis=-1, keepdims=True)

    BRN = 16
    row_spec = pl.BlockSpec((BRN, V), lambda i: (i, 0))
    col_spec = pl.BlockSpec((BRN, 1), lambda i: (i, 0))
    return pl.pallas_call(
        _nucleus_body,
        grid=(B // BRN,),
        in_specs=[row_spec, col_spec, col_spec],
        out_specs=row_spec,
        out_shape=jax.ShapeDtypeStruct((B, V), jnp.float32),
        scratch_shapes=[pltpu.VMEM((BRN, V), jnp.float32)],
    )(p, T, s1)


# speedup vs baseline: 61.8351x; 1.2259x over previous
import functools

import jax
import jax.numpy as jnp
from jax import lax
from jax.experimental import pallas as pl
from jax.experimental.pallas import tpu as pltpu
from jax.experimental.pallas import tpu_sc as plsc

_TEMP = 0.7
_TOPP = 0.9
_EPS = 1e-20
_K = 50
_INTMAX = 2147483647


_V = 100000
_NG = 250
_GV = 25
_CCAP = 2048


def _sc_topk_body(probs_hbm, out_hbm, rowbuf, gmxv, kthbuf, cbuf, gmx_smem):
    c = lax.axis_index("c")
    s = lax.axis_index("s")
    wid = s * 2 + c
    lane = lax.iota(jnp.int32, 16)
    kth = jnp.zeros((16,), jnp.float32)

    for j in range(2):
        row = wid * 2 + j
        pltpu.sync_copy(probs_hbm.at[row], rowbuf)

        def ga(g, carry):
            maxacc, gvec = carry
            base = g * (_GV * 16)
            gm = plsc.bitcast(rowbuf[pl.ds(base, 16)], jnp.int32)
            for jj in range(1, _GV):
                v = plsc.bitcast(rowbuf[pl.ds(base + jj * 16, 16)], jnp.int32)
                gm = jnp.maximum(gm, v)
            gs = jnp.max(gm)
            gmx_smem[g] = gs
            gvec = jnp.where(lane == g % 16, gs, gvec)

            @pl.when(g % 16 == 15)
            def _():
                gmxv[pl.ds((g // 16) * 16, 16)] = gvec

            return jnp.maximum(maxacc, gm), gvec

        maxacc, gvec = lax.fori_loop(
            0, _NG, ga,
            (jnp.full((16,), -2**31, jnp.int32), jnp.zeros((16,), jnp.int32)))
        gmxv[pl.ds((_NG // 16) * 16, 16)] = jnp.where(
            lane < _NG % 16, gvec, 0)
        m = jnp.max(maxacc)

        def b_cond(cs):
            lo, hi = cs
            return hi - lo > 1

        def b_body(cs):
            lo, hi = cs
            mid = lo + (hi - lo + 1) // 2
            vmid = jnp.full((16,), mid, jnp.int32)

            def cb(i, acc):
                return acc + (gmxv[pl.ds(i * 16, 16)] >= vmid).astype(jnp.int32)

            acc = lax.fori_loop(0, 16, cb, jnp.zeros((16,), jnp.int32))
            ok = jnp.sum(acc) >= _K
            return jnp.where(ok, mid, lo), jnp.where(ok, hi, mid)

        c50, _ = lax.while_loop(b_cond, b_body, (jnp.int32(0), m + 1))

        vc50 = jnp.full((16,), c50, jnp.int32)

        def cg(g, carry):
            off, ovf = carry

            def do(o):
                base = g * (_GV * 16)
                for jj in range(_GV):
                    v = plsc.bitcast(
                        rowbuf[pl.ds(base + jj * 16, 16)], jnp.int32)
                    msk = v >= vc50
                    plsc.store_compressed(cbuf.at[pl.ds(o, 16)], v, mask=msk)
                    o = o + plsc.all_reduce_population_count(msk)[0]
                return o

            qual = gmx_smem[g] >= c50
            can = off <= _CCAP - _GV * 16
            off = lax.cond(qual & can, do, lambda o: o, off)
            return off, ovf | (qual & jnp.logical_not(can))

        n, ovf = lax.fori_loop(0, _NG, cg, (jnp.int32(0), False))
        cbuf[pl.ds(n, 16)] = jnp.zeros((16,), jnp.int32)

        def f_compact(cs):
            lo, hi = cs
            mid = lo + (hi - lo + 1) // 2
            vmid = jnp.full((16,), mid, jnp.int32)

            def cb2(i, acc):
                return acc + (cbuf[pl.ds(i * 16, 16)] >= vmid).astype(jnp.int32)

            acc = lax.fori_loop(0, (n + 15) // 16, cb2,
                                jnp.zeros((16,), jnp.int32))
            ok = jnp.sum(acc) >= _K
            return jnp.where(ok, mid, lo), jnp.where(ok, hi, mid)

        def f_dense(cs):
            lo, hi = cs
            mid = lo + (hi - lo + 1) // 2
            vmid = jnp.full((16,), mid, jnp.int32)

            def fb(g, acc):
                def count_group(a):
                    base = g * (_GV * 16)
                    for jj in range(_GV):
                        v = plsc.bitcast(
                            rowbuf[pl.ds(base + jj * 16, 16)], jnp.int32)
                        a = a + (v >= vmid).astype(jnp.int32)
                    return a

                return lax.cond(gmx_smem[g] >= mid, count_group,
                                lambda a: a, acc)

            acc = lax.fori_loop(0, _NG, fb, jnp.zeros((16,), jnp.int32))
            ok = jnp.sum(acc) >= _K
            return jnp.where(ok, mid, lo), jnp.where(ok, hi, mid)

        kb = lax.cond(
            ovf,
            lambda: lax.while_loop(b_cond, f_dense, (c50, m + 1))[0],
            lambda: lax.while_loop(b_cond, f_compact, (c50, m + 1))[0])
        kfv = plsc.bitcast(jnp.full((16,), kb, jnp.int32), jnp.float32)
        kth = jnp.where(lane == j, kfv, kth)

    kthbuf[...] = kth
    pltpu.sync_copy(kthbuf, out_hbm.at[wid])


def _sc_topk(probs):
    mesh = plsc.VectorSubcoreMesh(core_axis_name="c", subcore_axis_name="s")
    fn = functools.partial(
        pl.kernel,
        mesh=mesh,
        compiler_params=pltpu.CompilerParams(needs_layout_passes=False),
        out_type=jax.ShapeDtypeStruct((32, 16), jnp.float32),
        scratch_types=[
            pltpu.VMEM((_V,), jnp.float32),
            pltpu.VMEM((256,), jnp.int32),
            pltpu.VMEM((16,), jnp.float32),
            pltpu.VMEM((_CCAP + 16,), jnp.int32),
            pltpu.SMEM((256,), jnp.int32),
        ],
    )(_sc_topk_body)
    return fn(probs)


def _nucleus_body(p_ref, kth_ref, s1_ref, out_ref, p1_ref):
    p = p_ref[...]
    kth = kth_ref[...]
    s1 = s1_ref[...]
    p1 = jnp.where(p >= kth, p, 0.0) / s1
    p1_ref[...] = p1
    xi = lax.bitcast_convert_type(p1, jnp.int32)
    ximax = jnp.max(xi, axis=-1, keepdims=True)
    minpos = jnp.min(jnp.where(xi > 0, xi, _INTMAX), axis=-1, keepdims=True)

    def cond(c):
        lo, hi = c
        return jnp.any(hi - lo > 1)

    def body(c):
        lo, hi = c
        mid = lo + lax.div(hi - lo + 1, 2)
        sge = jnp.sum(jnp.where(xi >= mid, p1, 0.0), axis=-1, keepdims=True)
        ok = sge <= _TOPP
        return jnp.where(ok, lo, mid), jnp.where(ok, mid, hi)

    _, tstar = lax.while_loop(cond, body, (minpos - 1, ximax + 1))

    vc0 = jnp.min(jnp.where(xi >= tstar, xi, _INTMAX), axis=-1, keepdims=True)
    vc = jnp.where(vc0 == _INTMAX, ximax, vc0)
    vd = jnp.max(jnp.where((xi > 0) & (xi < vc), xi, 0), axis=-1, keepdims=True)
    sgt_d = jnp.sum(jnp.where(xi > vd, p1, 0.0), axis=-1, keepdims=True)
    qd = jnp.max(jnp.where(xi == vd, p1, 0.0), axis=-1, keepdims=True)
    thb = jnp.where(sgt_d + qd <= _TOPP, vd, vc)

    keep = xi >= thb
    s2 = jnp.sum(jnp.where(keep, p1, 0.0), axis=-1, keepdims=True)
    out_ref[...] = jnp.where(keep, p1 / s2, 0.0)


def kernel(probs, k):
    del k
    B, V = probs.shape

    kthw = _sc_topk(probs)
    kthp = kthw[:, :2].reshape(B, 1)

    logits = jnp.log(probs + _EPS)
    logits = logits / _TEMP
    p = jax.nn.softmax(logits, axis=-1)

    T = jnp.max(jnp.where(probs == kthp, p, jnp.zeros_like(p)),
                axis=-1, keepdims=True)
    s1 = jnp.sum(jnp.where(p >= T, p, jnp.zeros_like(p)),
                 axis=-1, keepdims=True)

    BRN = 16
    row_spec = pl.BlockSpec((BRN, V), lambda i: (i, 0))
    col_spec = pl.BlockSpec((BRN, 1), lambda i: (i, 0))
    return pl.pallas_call(
        _nucleus_body,
        grid=(B // BRN,),
        in_specs=[row_spec, col_spec, col_spec],
        out_specs=row_spec,
        out_shape=jax.ShapeDtypeStruct((B, V), jnp.float32),
        scratch_shapes=[pltpu.VMEM((BRN, V), jnp.float32)],
    )(p, T, s1)


# speedup vs baseline: 84.6765x; 1.3694x over previous
import functools

import jax
import jax.numpy as jnp
from jax import lax
from jax.experimental import pallas as pl
from jax.experimental.pallas import tpu as pltpu
from jax.experimental.pallas import tpu_sc as plsc

_TEMP = 0.7
_TOPP = 0.9
_EPS = 1e-20
_K = 50
_INTMAX = 2147483647


_V = 100000
_NG = 250
_GV = 25
_CCAP = 2048
_KC = 64


def _sc_topk_body(probs_hbm, kth_hbm, idx_hbm, nk_hbm, rowbuf, gmxv, kthbuf,
                  cbuf, cibuf, idxbuf, nkbuf, gmx_smem):
    c = lax.axis_index("c")
    s = lax.axis_index("s")
    wid = s * 2 + c
    lane = lax.iota(jnp.int32, 16)
    kth = jnp.zeros((16,), jnp.float32)
    nkv = jnp.zeros((16,), jnp.int32)

    for j in range(2):
        row = wid * 2 + j
        pltpu.sync_copy(probs_hbm.at[row], rowbuf)

        def ga(g, carry):
            maxacc, gvec = carry
            base = g * (_GV * 16)
            gm = plsc.bitcast(rowbuf[pl.ds(base, 16)], jnp.int32)
            for jj in range(1, _GV):
                v = plsc.bitcast(rowbuf[pl.ds(base + jj * 16, 16)], jnp.int32)
                gm = jnp.maximum(gm, v)
            gs = jnp.max(gm)
            gmx_smem[g] = gs
            gvec = jnp.where(lane == g % 16, gs, gvec)

            @pl.when(g % 16 == 15)
            def _():
                gmxv[pl.ds((g // 16) * 16, 16)] = gvec

            return jnp.maximum(maxacc, gm), gvec

        maxacc, gvec = lax.fori_loop(
            0, _NG, ga,
            (jnp.full((16,), -2**31, jnp.int32), jnp.zeros((16,), jnp.int32)))
        gmxv[pl.ds((_NG // 16) * 16, 16)] = jnp.where(
            lane < _NG % 16, gvec, 0)
        m = jnp.max(maxacc)

        def b_cond(cs):
            lo, hi = cs
            return hi - lo > 1

        def b_body(cs):
            lo, hi = cs
            mid = lo + (hi - lo + 1) // 2
            vmid = jnp.full((16,), mid, jnp.int32)

            def cb(i, acc):
                return acc + (gmxv[pl.ds(i * 16, 16)] >= vmid).astype(jnp.int32)

            acc = lax.fori_loop(0, 16, cb, jnp.zeros((16,), jnp.int32))
            ok = jnp.sum(acc) >= _K
            return jnp.where(ok, mid, lo), jnp.where(ok, hi, mid)

        c50, _ = lax.while_loop(b_cond, b_body, (jnp.int32(0), m + 1))

        vc50 = jnp.full((16,), c50, jnp.int32)

        def cg(g, carry):
            off, ovf = carry

            def do(o):
                base = g * (_GV * 16)
                for jj in range(_GV):
                    v = plsc.bitcast(
                        rowbuf[pl.ds(base + jj * 16, 16)], jnp.int32)
                    msk = v >= vc50
                    plsc.store_compressed(cbuf.at[pl.ds(o, 16)], v, mask=msk)
                    iv = jnp.full((16,), base + jj * 16, jnp.int32) + lane
                    plsc.store_compressed(cibuf.at[pl.ds(o, 16)], iv, mask=msk)
                    o = o + plsc.all_reduce_population_count(msk)[0]
                return o

            qual = gmx_smem[g] >= c50
            can = off <= _CCAP - _GV * 16
            off = lax.cond(qual & can, do, lambda o: o, off)
            return off, ovf | (qual & jnp.logical_not(can))

        n, ovf = lax.fori_loop(0, _NG, cg, (jnp.int32(0), False))
        cbuf[pl.ds(n, 16)] = jnp.zeros((16,), jnp.int32)

        def f_compact(cs):
            lo, hi = cs
            mid = lo + (hi - lo + 1) // 2
            vmid = jnp.full((16,), mid, jnp.int32)

            def cb2(i, acc):
                return acc + (cbuf[pl.ds(i * 16, 16)] >= vmid).astype(jnp.int32)

            acc = lax.fori_loop(0, (n + 15) // 16, cb2,
                                jnp.zeros((16,), jnp.int32))
            ok = jnp.sum(acc) >= _K
            return jnp.where(ok, mid, lo), jnp.where(ok, hi, mid)

        def f_dense(cs):
            lo, hi = cs
            mid = lo + (hi - lo + 1) // 2
            vmid = jnp.full((16,), mid, jnp.int32)

            def fb(g, acc):
                def count_group(a):
                    base = g * (_GV * 16)
                    for jj in range(_GV):
                        v = plsc.bitcast(
                            rowbuf[pl.ds(base + jj * 16, 16)], jnp.int32)
                        a = a + (v >= vmid).astype(jnp.int32)
                    return a

                return lax.cond(gmx_smem[g] >= mid, count_group,
                                lambda a: a, acc)

            acc = lax.fori_loop(0, _NG, fb, jnp.zeros((16,), jnp.int32))
            ok = jnp.sum(acc) >= _K
            return jnp.where(ok, mid, lo), jnp.where(ok, hi, mid)

        kb = lax.cond(
            ovf,
            lambda: lax.while_loop(b_cond, f_dense, (c50, m + 1))[0],
            lambda: lax.while_loop(b_cond, f_compact, (c50, m + 1))[0])
        kfv = plsc.bitcast(jnp.full((16,), kb, jnp.int32), jnp.float32)
        kth = jnp.where(lane == j, kfv, kth)

        for t in range(_KC // 16 + 1):
            idxbuf[pl.ds(t * 16, 16)] = jnp.zeros((16,), jnp.int32)
        vkb = jnp.full((16,), kb, jnp.int32)

        def fe(i, o):
            v = cbuf[pl.ds(i * 16, 16)]
            iv = cibuf[pl.ds(i * 16, 16)]
            msk = v >= vkb

            @pl.when(o < _KC)
            def _():
                plsc.store_compressed(idxbuf.at[pl.ds(o, 16)], iv, mask=msk)

            return o + plsc.all_reduce_population_count(msk)[0]

        nk0 = lax.fori_loop(0, (n + 15) // 16, fe, jnp.int32(0))
        nk = jnp.where(ovf | (nk0 > _KC), -1, nk0)
        nkv = jnp.where(lane == j, nk, nkv)
        pltpu.sync_copy(idxbuf.at[pl.ds(0, _KC)],
                        idx_hbm.at[pl.ds(row * _KC, _KC)])

    kthbuf[...] = kth
    pltpu.sync_copy(kthbuf, kth_hbm.at[wid])
    nkbuf[...] = nkv
    pltpu.sync_copy(nkbuf, nk_hbm.at[pl.ds(wid * 16, 16)])


def _sc_topk(probs):
    mesh = plsc.VectorSubcoreMesh(core_axis_name="c", subcore_axis_name="s")
    fn = functools.partial(
        pl.kernel,
        mesh=mesh,
        compiler_params=pltpu.CompilerParams(needs_layout_passes=False),
        out_type=[
            jax.ShapeDtypeStruct((32, 16), jnp.float32),
            jax.ShapeDtypeStruct((64 * _KC,), jnp.int32),
            jax.ShapeDtypeStruct((512,), jnp.int32),
        ],
        scratch_types=[
            pltpu.VMEM((_V,), jnp.float32),
            pltpu.VMEM((256,), jnp.int32),
            pltpu.VMEM((16,), jnp.float32),
            pltpu.VMEM((_CCAP + 16,), jnp.int32),
            pltpu.VMEM((_CCAP + 16,), jnp.int32),
            pltpu.VMEM((_KC + 16,), jnp.int32),
            pltpu.VMEM((16,), jnp.int32),
            pltpu.SMEM((256,), jnp.int32),
        ],
    )(_sc_topk_body)
    return fn(probs)


def _nucleus_thb(xi, p1, minpos, ximax, lo0, hi0):
    def cond(c):
        lo, hi = c
        return jnp.any(hi - lo > 1)

    def body(c):
        lo, hi = c
        mid = lo + lax.div(hi - lo + 1, 2)
        sge = jnp.sum(jnp.where(xi >= mid, p1, 0.0), axis=-1, keepdims=True)
        ok = (sge <= _TOPP) & (mid > minpos)
        return jnp.where(ok, lo, mid), jnp.where(ok, mid, hi)

    _, tstar = lax.while_loop(cond, body, (lo0, hi0))

    vc0 = jnp.min(jnp.where(xi >= tstar, xi, _INTMAX), axis=-1, keepdims=True)
    vc = jnp.where(vc0 == _INTMAX, ximax, vc0)
    vd = jnp.max(jnp.where((xi > 0) & (xi < vc), xi, 0), axis=-1, keepdims=True)
    sgt_d = jnp.sum(jnp.where(xi > vd, p1, 0.0), axis=-1, keepdims=True)
    qd = jnp.max(jnp.where(xi == vd, p1, 0.0), axis=-1, keepdims=True)
    return jnp.where(sgt_d + qd <= _TOPP, vd, vc)


def _emit_body(p_ref, pc_ref, kth_ref, s1_ref, nk_ref, out_ref, p1_ref):
    p = p_ref[...]
    pc = pc_ref[...]
    kth = kth_ref[...]
    s1 = s1_ref[...]
    nk = nk_ref[...]
    p1 = jnp.where(p >= kth, p, 0.0) / s1
    p1_ref[...] = p1
    xid = lax.bitcast_convert_type(p1, jnp.int32)

    good = nk >= 0
    slot = lax.broadcasted_iota(jnp.int32, pc.shape, 1)
    valid = slot < jnp.where(good, nk, 0)
    p1c = jnp.where(valid, pc, 0.0) / s1
    xic = lax.bitcast_convert_type(p1c, jnp.int32)
    ximax_c = jnp.max(xic, axis=-1, keepdims=True)
    minpos_c = jnp.min(jnp.where(xic > 0, xic, _INTMAX), axis=-1,
                       keepdims=True)
    thb_c = _nucleus_thb(xic, p1c, minpos_c, ximax_c,
                         minpos_c - 1, ximax_c + 1)

    def dense_fn():
        ximax_d = jnp.max(xid, axis=-1, keepdims=True)
        minpos_d = jnp.min(jnp.where(xid > 0, xid, _INTMAX), axis=-1,
                           keepdims=True)
        lo0 = jnp.where(good, ximax_d, minpos_d - 1)
        return _nucleus_thb(xid, p1, minpos_d, ximax_d, lo0, ximax_d + 1)

    thb_d = lax.cond(jnp.any(jnp.logical_not(good)), dense_fn,
                     lambda: jnp.zeros_like(thb_c))
    thb = jnp.where(good, thb_c, thb_d)

    keep = xid >= thb
    s2 = jnp.sum(jnp.where(keep, p1, 0.0), axis=-1, keepdims=True)
    out_ref[...] = jnp.where(keep, p1 / s2, 0.0)


def kernel(probs, k):
    del k
    B, V = probs.shape

    kthw, idxf, nkf = _sc_topk(probs)
    kthp = kthw[:, :2].reshape(B, 1)
    idxs = idxf.reshape(B, _KC)
    nk = nkf.reshape(32, 16)[:, :2].reshape(B, 1)

    logits = jnp.log(probs + _EPS)
    logits = logits / _TEMP
    p = jax.nn.softmax(logits, axis=-1)

    T = jnp.max(jnp.where(probs == kthp, p, jnp.zeros_like(p)),
                axis=-1, keepdims=True)
    s1 = jnp.sum(jnp.where(p >= T, p, jnp.zeros_like(p)),
                 axis=-1, keepdims=True)

    pc = jnp.take_along_axis(p, idxs, axis=1)

    BRN = 16
    row_spec = pl.BlockSpec((BRN, V), lambda i: (i, 0))
    col_spec = pl.BlockSpec((BRN, 1), lambda i: (i, 0))
    return pl.pallas_call(
        _emit_body,
        grid=(B // BRN,),
        in_specs=[row_spec, pl.BlockSpec((BRN, _KC), lambda i: (i, 0)),
                  col_spec, col_spec, col_spec],
        out_specs=row_spec,
        out_shape=jax.ShapeDtypeStruct((B, V), jnp.float32),
        scratch_shapes=[pltpu.VMEM((BRN, V), jnp.float32)],
    )(p, pc, T, s1, nk)
